# speedup vs baseline: 1.2220x; 1.2220x over previous
import functools

import jax
import jax.numpy as jnp
from jax import lax
from jax.experimental import pallas as pl
from jax.experimental.pallas import tpu as pltpu
from jax.experimental.pallas import tpu_sc as plsc

_B = 16384
_F = 26
_FIELD = 100000
_NC = 2
_NS = 16
_NW = _NC * _NS
_BPW = _B // _NW
_IPW = _F * _BPW
_CH = 128
_NCH = _IPW // _CH
_L = 16


def _fm_body(xprep, table, out, xt_v, idx_v, rows_v, acc_v, sem):
    wid = lax.axis_index("s") * _NC + lax.axis_index("c")

    pltpu.sync_copy(xprep.at[wid], xt_v)

    def idx_chunk(jc, _):
        off = (jc // 4) * _FIELD

        def idx_vec(j, _):
            v = xt_v[pl.ds(jc * _CH + j * _L, _L)] + off
            idx_v[jc, pl.ds(j * _L, _L)] = v
            return 0

        return lax.fori_loop(0, _CH // _L, idx_vec, 0)

    lax.fori_loop(0, _NCH, idx_chunk, 0)

    def fire(jc, _):
        pltpu.async_copy(
            table.at[idx_v.at[jc]], rows_v.at[pl.ds(jc * _CH, _CH)], sem
        )
        return 0

    lax.fori_loop(0, _NCH, fire, 0)

    def drain(jc, _):
        pltpu.make_async_copy(
            table.at[idx_v.at[jc]], rows_v.at[pl.ds(jc * _CH, _CH)], sem
        ).wait()
        return 0

    lax.fori_loop(0, _NCH, drain, 0)

    def zero(o, _):
        acc_v[pl.ds(o * _L, _L)] = jnp.zeros((_L,), jnp.float32)
        return 0

    lax.fori_loop(0, _BPW // _L, zero, 0)

    def acc_chunk(jc, _):
        b0 = (jc % 4) * _CH

        def acc_vec(j, _):
            s = pl.ds(b0 + j * _L, _L)
            acc_v[s] = acc_v[s] + rows_v[pl.ds(jc * _CH + j * _L, _L)]
            return 0

        return lax.fori_loop(0, _CH // _L, acc_vec, 0)

    lax.fori_loop(0, _NCH, acc_chunk, 0)

    pltpu.sync_copy(acc_v, out.at[pl.ds(wid * _BPW, _BPW)])


@jax.jit
def _fm(xprep, table_flat):
    mesh = plsc.VectorSubcoreMesh(core_axis_name="c", subcore_axis_name="s")
    return pl.kernel(
        _fm_body,
        mesh=mesh,
        out_type=jax.ShapeDtypeStruct((_B,), jnp.float32),
        scratch_types=[
            pltpu.VMEM((_IPW,), jnp.int32),
            pltpu.VMEM((_NCH, _CH), jnp.int32),
            pltpu.VMEM((_IPW,), jnp.float32),
            pltpu.VMEM((_BPW,), jnp.float32),
            pltpu.SemaphoreType.DMA,
        ],
    )(xprep, table_flat)


def kernel(x, table):
    xprep = jnp.transpose(x.reshape(_NW, _BPW, _F), (0, 2, 1)).reshape(_NW, _IPW)
    out = _fm(xprep, table.reshape(-1))
    return out.reshape(_B, 1)


# speedup vs baseline: 4.2294x; 3.4609x over previous
import functools

import jax
import jax.numpy as jnp
from jax import lax
from jax.experimental import pallas as pl
from jax.experimental.pallas import tpu as pltpu
from jax.experimental.pallas import tpu_sc as plsc

_B = 16384
_F = 26
_FIELD = 100000
_NC = 2
_NS = 16
_NW = _NC * _NS
_BPW = _B // _NW
_IPW = _F * _BPW
_CH = 128
_NCH = _IPW // _CH
_L = 16


def _fm_body(xprep, table, out, xt_v, idx_v, rows_v, acc_v, sem):
    wid = lax.axis_index("s") * _NC + lax.axis_index("c")

    pltpu.sync_copy(xprep.at[wid], xt_v)

    def idx_chunk(jc, _):
        off = (jc // 4) * _FIELD

        def idx_vec(j, _):
            v = xt_v[pl.ds(jc * _CH + j * _L, _L)] + off
            idx_v[jc, pl.ds(j * _L, _L)] = v
            return 0

        return lax.fori_loop(0, _CH // _L, idx_vec, 0)

    lax.fori_loop(0, _NCH, idx_chunk, 0)

    def fire(jc, _):
        pltpu.async_copy(
            table.at[idx_v.at[pl.ds(jc, 1)]], rows_v.at[pl.ds(jc, 1)], sem
        )
        return 0

    lax.fori_loop(0, _NCH, fire, 0)

    def drain(jc, _):
        pltpu.make_async_copy(
            table.at[idx_v.at[pl.ds(jc, 1)]], rows_v.at[pl.ds(jc, 1)], sem
        ).wait()
        return 0

    lax.fori_loop(0, _NCH, drain, 0)

    def zero(o, _):
        acc_v[pl.ds(o * _L, _L)] = jnp.zeros((_L,), jnp.float32)
        return 0

    lax.fori_loop(0, _BPW // _L, zero, 0)

    def acc_chunk(jc, _):
        b0 = (jc % 4) * _CH

        def acc_vec(j, _):
            s = pl.ds(b0 + j * _L, _L)
            acc_v[s] = acc_v[s] + rows_v[jc, pl.ds(j * _L, _L)]
            return 0

        return lax.fori_loop(0, _CH // _L, acc_vec, 0)

    lax.fori_loop(0, _NCH, acc_chunk, 0)

    pltpu.sync_copy(acc_v, out.at[pl.ds(wid * _BPW, _BPW)])


@jax.jit
def _fm(xprep, table2d):
    mesh = plsc.VectorSubcoreMesh(
        core_axis_name="c", subcore_axis_name="s", num_cores=_NC
    )
    return pl.kernel(
        _fm_body,
        mesh=mesh,
        out_type=jax.ShapeDtypeStruct((_B,), jnp.float32),
        scratch_types=[
            pltpu.VMEM((_IPW,), jnp.int32),
            pltpu.VMEM((_NCH, _CH), jnp.int32),
            pltpu.VMEM((_NCH, _CH), jnp.float32),
            pltpu.VMEM((_BPW,), jnp.float32),
            pltpu.SemaphoreType.DMA,
        ],
    )(xprep, table2d)


def kernel(x, table):
    xprep = jnp.transpose(x.reshape(_NW, _BPW, _F), (0, 2, 1)).reshape(_NW, _IPW)
    out = _fm(xprep, table.reshape(1, _F * _FIELD))
    return out.reshape(_B, 1)


# speedup vs baseline: 4.8500x; 1.1468x over previous
import functools

import jax
import jax.numpy as jnp
from jax import lax
from jax.experimental import pallas as pl
from jax.experimental.pallas import tpu as pltpu
from jax.experimental.pallas import tpu_sc as plsc

_B = 16384
_F = 26
_FIELD = 100000
_NC = 2
_NS = 16
_NW = _NC * _NS
_BPW = _B // _NW
_IPW = _F * _BPW
_CH = 128
_NG = 4
_GSZ = _F * _CH
_L = 16
_J = _CH // _L


def _fm_body(xr, table, out, xt_v, idx_v, rows_v, acc_v, s0, s1, s2, s3):
    wid = lax.axis_index("s") * _NC + lax.axis_index("c")
    sems = [s0, s1, s2, s3]
    table1d = table.at[0]

    pltpu.sync_copy(xr.at[wid], xt_v)

    for c in range(_NG):

        def idx_row(f, _, c=c):
            off = f * _FIELD
            for j in range(_J):
                v = xt_v[pl.ds(f * _BPW + c * _CH + j * _L, _L)]
                idx_v[pl.ds(c * _GSZ + f * _CH + j * _L, _L)] = v + off
            return 0

        lax.fori_loop(0, _F, idx_row, 0)
        pltpu.async_copy(
            table1d.at[idx_v.at[pl.ds(c * _GSZ, _GSZ)]],
            rows_v.at[pl.ds(c * _GSZ, _GSZ)],
            sems[c],
        )

    for c in range(_NG):
        pltpu.make_async_copy(
            table1d.at[idx_v.at[pl.ds(c * _GSZ, _GSZ)]],
            rows_v.at[pl.ds(c * _GSZ, _GSZ)],
            sems[c],
        ).wait()

        def red(j, _, c=c):
            base = c * _GSZ + j * _L
            a0 = rows_v[pl.ds(base, _L)]
            a1 = rows_v[pl.ds(base + _CH, _L)]
            for f in range(2, _F, 2):
                a0 = a0 + rows_v[pl.ds(base + f * _CH, _L)]
                a1 = a1 + rows_v[pl.ds(base + (f + 1) * _CH, _L)]
            acc_v[pl.ds(c * _CH + j * _L, _L)] = a0 + a1
            return 0

        lax.fori_loop(0, _J, red, 0)

    pltpu.sync_copy(acc_v, out.at[pl.ds(wid * _BPW, _BPW)])


@jax.jit
def _fm(xr, table2d):
    mesh = plsc.VectorSubcoreMesh(
        core_axis_name="c", subcore_axis_name="s", num_cores=_NC
    )
    return pl.kernel(
        _fm_body,
        mesh=mesh,
        out_type=jax.ShapeDtypeStruct((_B,), jnp.float32),
        scratch_types=[
            pltpu.VMEM((_IPW,), jnp.int32),
            pltpu.VMEM((_IPW,), jnp.int32),
            pltpu.VMEM((_IPW,), jnp.float32),
            pltpu.VMEM((_BPW,), jnp.float32),
            pltpu.SemaphoreType.DMA,
            pltpu.SemaphoreType.DMA,
            pltpu.SemaphoreType.DMA,
            pltpu.SemaphoreType.DMA,
        ],
    )(xr, table2d)


def kernel(x, table):
    xprep = jnp.transpose(x.reshape(_NW, _BPW, _F), (0, 2, 1)).reshape(_NW, _IPW)
    out = _fm(xprep, table.reshape(1, _F * _FIELD))
    return out.reshape(_B, 1)


# speedup vs baseline: 4.9060x; 1.0115x over previous
import functools

import jax
import jax.numpy as jnp
from jax import lax
from jax.experimental import pallas as pl
from jax.experimental.pallas import tpu as pltpu
from jax.experimental.pallas import tpu_sc as plsc

_B = 16384
_F = 26
_FIELD = 100000
_NC = 2
_NS = 16
_NW = _NC * _NS
_BPW = _B // _NW
_IPW = _F * _BPW
_L = 16
_JW = _BPW // _L
_GROUPS = ((0, 7), (7, 14), (14, 20), (20, 26))


def _fm_body(xr, table, out, xt_v, rows_v, acc_v, s0, s1, s2, s3):
    wid = lax.axis_index("s") * _NC + lax.axis_index("c")
    sems = [s0, s1, s2, s3]
    table1d = table.at[0]

    pltpu.sync_copy(xr.at[wid], xt_v)

    for g, (f0, f1) in enumerate(_GROUPS):
        for f in range(f0, f1):
            pltpu.async_copy(
                table1d.at[pl.ds(f * _FIELD, _FIELD)].at[
                    xt_v.at[pl.ds(f * _BPW, _BPW)]
                ],
                rows_v.at[pl.ds(f * _BPW, _BPW)],
                sems[g],
            )

    for g, (f0, f1) in enumerate(_GROUPS):
        for f in range(f0, f1):
            pltpu.make_async_copy(
                table1d.at[pl.ds(f * _FIELD, _FIELD)].at[
                    xt_v.at[pl.ds(f * _BPW, _BPW)]
                ],
                rows_v.at[pl.ds(f * _BPW, _BPW)],
                sems[g],
            ).wait()

        def red(j, _, f0=f0, f1=f1, first=(g == 0)):
            a0 = rows_v[pl.ds(f0 * _BPW + j * _L, _L)]
            a1 = rows_v[pl.ds((f0 + 1) * _BPW + j * _L, _L)]
            for f in range(f0 + 2, f1, 2):
                a0 = a0 + rows_v[pl.ds(f * _BPW + j * _L, _L)]
            for f in range(f0 + 3, f1, 2):
                a1 = a1 + rows_v[pl.ds(f * _BPW + j * _L, _L)]
            s = pl.ds(j * _L, _L)
            if first:
                acc_v[s] = a0 + a1
            else:
                acc_v[s] = acc_v[s] + (a0 + a1)
            return 0

        lax.fori_loop(0, _JW, red, 0)

    pltpu.sync_copy(acc_v, out.at[pl.ds(wid * _BPW, _BPW)])


@jax.jit
def _fm(xr, table2d):
    mesh = plsc.VectorSubcoreMesh(
        core_axis_name="c", subcore_axis_name="s", num_cores=_NC
    )
    return pl.kernel(
        _fm_body,
        mesh=mesh,
        out_type=jax.ShapeDtypeStruct((_B,), jnp.float32),
        scratch_types=[
            pltpu.VMEM((_IPW,), jnp.int32),
            pltpu.VMEM((_IPW,), jnp.float32),
            pltpu.VMEM((_BPW,), jnp.float32),
            pltpu.SemaphoreType.DMA,
            pltpu.SemaphoreType.DMA,
            pltpu.SemaphoreType.DMA,
            pltpu.SemaphoreType.DMA,
        ],
    )(xr, table2d)


def kernel(x, table):
    xprep = jnp.transpose(x.reshape(_NW, _BPW, _F), (0, 2, 1)).reshape(_NW, _IPW)
    out = _fm(xprep, table.reshape(1, _F * _FIELD))
    return out.reshape(_B, 1)


# speedup vs baseline: 4.9290x; 1.0047x over previous
import functools

import jax
import jax.numpy as jnp
from jax import lax
from jax.experimental import pallas as pl
from jax.experimental.pallas import tpu as pltpu
from jax.experimental.pallas import tpu_sc as plsc

_B = 16384
_F = 26
_FIELD = 100000
_NC = 2
_NS = 16
_NW = _NC * _NS
_BPW = _B // _NW
_IPW = _F * _BPW
_L = 16
_JW = _BPW // _L
_GROUPS = ((0, 7), (7, 14), (14, 20), (20, 26))


def _fm_body(xr, table, out, xt_v, rows_v, acc_v, s0, s1, s2, s3):
    wid = lax.axis_index("s") * _NC + lax.axis_index("c")
    sems = [s0, s1, s2, s3]
    table1d = table.at[0]

    pltpu.sync_copy(xr.at[wid], xt_v)

    for g, (f0, f1) in enumerate(_GROUPS):

        def fire(f, _, g=g):
            pltpu.async_copy(
                table1d.at[pl.ds(f * _FIELD, _FIELD)].at[
                    xt_v.at[pl.ds(f * _BPW, _BPW)]
                ],
                rows_v.at[pl.ds(f * _BPW, _BPW)],
                sems[g],
            )
            return 0

        lax.fori_loop(f0, f1, fire, 0)

    for g, (f0, f1) in enumerate(_GROUPS):
        gsz = (f1 - f0) * _BPW
        pltpu.make_async_copy(
            table1d.at[pl.ds(0, gsz)],
            rows_v.at[pl.ds(f0 * _BPW, gsz)],
            sems[g],
        ).wait()

        def red(j, _, f0=f0, f1=f1, first=(g == 0)):
            a0 = rows_v[pl.ds(f0 * _BPW + j * _L, _L)]
            a1 = rows_v[pl.ds((f0 + 1) * _BPW + j * _L, _L)]
            for f in range(f0 + 2, f1, 2):
                a0 = a0 + rows_v[pl.ds(f * _BPW + j * _L, _L)]
            for f in range(f0 + 3, f1, 2):
                a1 = a1 + rows_v[pl.ds(f * _BPW + j * _L, _L)]
            s = pl.ds(j * _L, _L)
            if first:
                acc_v[s] = a0 + a1
            else:
                acc_v[s] = acc_v[s] + (a0 + a1)
            return 0

        lax.fori_loop(0, _JW, red, 0)

    pltpu.sync_copy(acc_v, out.at[pl.ds(wid * _BPW, _BPW)])


@jax.jit
def _fm(xr, table2d):
    mesh = plsc.VectorSubcoreMesh(
        core_axis_name="c", subcore_axis_name="s", num_cores=_NC
    )
    return pl.kernel(
        _fm_body,
        mesh=mesh,
        out_type=jax.ShapeDtypeStruct((_B,), jnp.float32),
        scratch_types=[
            pltpu.VMEM((_IPW,), jnp.int32),
            pltpu.VMEM((_IPW,), jnp.float32),
            pltpu.VMEM((_BPW,), jnp.float32),
            pltpu.SemaphoreType.DMA,
            pltpu.SemaphoreType.DMA,
            pltpu.SemaphoreType.DMA,
            pltpu.SemaphoreType.DMA,
        ],
    )(xr, table2d)


def kernel(x, table):
    xprep = jnp.transpose(x.reshape(_NW, _BPW, _F), (0, 2, 1)).reshape(_NW, _IPW)
    out = _fm(xprep, table.reshape(1, _F * _FIELD))
    return out.reshape(_B, 1)


# speedup vs baseline: 4.9745x; 1.0092x over previous
import functools

import jax
import jax.numpy as jnp
from jax import lax
from jax.experimental import pallas as pl
from jax.experimental.pallas import tpu as pltpu
from jax.experimental.pallas import tpu_sc as plsc

_B = 16384
_F = 26
_FIELD = 100000
_NC = 2
_NS = 16
_NW = _NC * _NS
_BPW = _B // _NW
_IPW = _F * _BPW
_L = 16
_JW = _BPW // _L
_GROUPS = ((0, 7), (7, 14), (14, 20), (20, 26))


def _fm_body(xr, table, out, xt_v, rows_v, acc_v, s0):
    wid = lax.axis_index("s") * _NC + lax.axis_index("c")
    table1d = table.at[0]

    pltpu.sync_copy(xr.at[wid], xt_v)

    def fire(f, _):
        pltpu.async_copy(
            table1d.at[pl.ds(f * _FIELD, _FIELD)].at[
                xt_v.at[pl.ds(f * _BPW, _BPW)]
            ],
            rows_v.at[pl.ds(f * _BPW, _BPW)],
            s0,
        )
        return 0

    lax.fori_loop(0, _F, fire, 0)

    pltpu.make_async_copy(table1d.at[pl.ds(0, _IPW)], rows_v, s0).wait()

    def red(j, _):
        a0 = rows_v[pl.ds(j * _L, _L)]
        a1 = rows_v[pl.ds(_BPW + j * _L, _L)]
        for f in range(2, _F, 2):
            a0 = a0 + rows_v[pl.ds(f * _BPW + j * _L, _L)]
        for f in range(3, _F, 2):
            a1 = a1 + rows_v[pl.ds(f * _BPW + j * _L, _L)]
        acc_v[pl.ds(j * _L, _L)] = a0 + a1
        return 0

    lax.fori_loop(0, _JW, red, 0)

    pltpu.sync_copy(acc_v, out.at[pl.ds(wid * _BPW, _BPW)])


@jax.jit
def _fm(xr, table2d):
    mesh = plsc.VectorSubcoreMesh(
        core_axis_name="c", subcore_axis_name="s", num_cores=_NC
    )
    return pl.kernel(
        _fm_body,
        mesh=mesh,
        out_type=jax.ShapeDtypeStruct((_B,), jnp.float32),
        scratch_types=[
            pltpu.VMEM((_IPW,), jnp.int32),
            pltpu.VMEM((_IPW,), jnp.float32),
            pltpu.VMEM((_BPW,), jnp.float32),
            pltpu.SemaphoreType.DMA,
        ],
    )(xr, table2d)


def kernel(x, table):
    xprep = jnp.transpose(x.reshape(_NW, _BPW, _F), (0, 2, 1)).reshape(_NW, _IPW)
    out = _fm(xprep, table.reshape(1, _F * _FIELD))
    return out.reshape(_B, 1)
